# baseline (speedup 1.0000x reference)
import functools

import jax
import jax.numpy as jnp
from jax import lax
from jax.experimental import pallas as pl
from jax.experimental.pallas import tpu as pltpu
from jax.experimental.pallas import tpu_sc as plsc

B = 16384
K = 32
NC, NS, L = 2, 16, 16
NW = NC * NS
BPW = B // NW
NCHUNK = 4
CH = BPW // NCHUNK
GRP = CH // L

_mesh = plsc.VectorSubcoreMesh(
    core_axis_name="c", subcore_axis_name="s", num_cores=NC, num_subcores=NS
)


def _body(uidx_hbm, iidx_hbm, uemb_hbm, iemb_hbm, out_hbm,
          uidx_v, iidx_v, urows_v, irows_v, out_v, sem):
    wid = lax.axis_index("s") * NC + lax.axis_index("c")

    pltpu.sync_copy(uidx_hbm.at[pl.ds(wid * NCHUNK, NCHUNK)], uidx_v)
    pltpu.sync_copy(iidx_hbm.at[pl.ds(wid * NCHUNK, NCHUNK)], iidx_v)

    copies = []
    for c in range(NCHUNK):
        copies.append(pltpu.async_copy(
            uemb_hbm.at[uidx_v.at[c]], urows_v.at[pl.ds(c * CH, CH)], sem))
        copies.append(pltpu.async_copy(
            iemb_hbm.at[iidx_v.at[c]], irows_v.at[pl.ds(c * CH, CH)], sem))

    iot = lax.iota(jnp.int32, L)

    for c in range(NCHUNK):
        copies[2 * c].wait()
        copies[2 * c + 1].wait()

        def g_body(g, carry, c=c):
            rows = iot + (c * CH + g * L)
            acc = jnp.zeros((L,), jnp.float32)
            for k in range(K):
                colk = jnp.full((L,), k, jnp.int32)
                u = plsc.load_gather(urows_v, [rows, colk])
                v = plsc.load_gather(irows_v, [rows, colk])
                acc = acc + u * v
            out_v[pl.ds(c * CH + g * L, L)] = acc
            return carry

        lax.fori_loop(0, GRP, g_body, 0)

    pltpu.sync_copy(out_v, out_hbm.at[pl.ds(wid * BPW, BPW)])


_kern = pl.kernel(
    _body,
    out_type=jax.ShapeDtypeStruct((B,), jnp.float32),
    mesh=_mesh,
    scratch_types=[
        pltpu.VMEM((NCHUNK, CH), jnp.int32),
        pltpu.VMEM((NCHUNK, CH), jnp.int32),
        pltpu.VMEM((BPW, K), jnp.float32),
        pltpu.VMEM((BPW, K), jnp.float32),
        pltpu.VMEM((BPW,), jnp.float32),
        pltpu.SemaphoreType.DMA,
    ],
    compiler_params=pltpu.CompilerParams(
        needs_layout_passes=False, use_tc_tiling_on_sc=False),
)


@jax.jit
def kernel(user_input, item_input, user_embedding, item_embedding):
    uidx = user_input.reshape(NW * NCHUNK, CH)
    iidx = item_input.reshape(NW * NCHUNK, CH)
    out = _kern(uidx, iidx, user_embedding, item_embedding)
    return out.reshape(B, 1)
